# baseline (speedup 1.0000x reference)
import functools

import jax
import jax.numpy as jnp
from jax import lax
from jax.experimental import pallas as pl
from jax.experimental.pallas import tpu as pltpu
from jax.experimental.pallas import tpu_sc as plsc

TM = 256
HB = 512
HC = 256


def _ffn_body(pt_ref, ex_ref, rs_ref, re_ref, first_ref,
              xs_ref, w1_ref, w3_ref, w2_ref, wrow_ref, out_ref, acc_ref, *,
              ht):
    h = pl.program_id(1)
    x = xs_ref[...]
    upd = None
    for c in range(0, HB, HC):
        g = jnp.dot(x, w1_ref[0, :, c:c + HC],
                    preferred_element_type=jnp.float32)
        u = jnp.dot(x, w3_ref[0, :, c:c + HC],
                    preferred_element_type=jnp.float32)
        mid = g * jax.nn.sigmoid(g) * u
        d = jnp.dot(mid, w2_ref[0, c:c + HC, :],
                    preferred_element_type=jnp.float32)
        upd = d if upd is None else upd + d

    @pl.when(h == 0)
    def _():
        acc_ref[...] = upd

    @pl.when(h != 0)
    def _():
        acc_ref[...] += upd

    @pl.when(h == ht - 1)
    def _():
        i = pl.program_id(0)
        rs = rs_ref[i]
        re = re_ref[i]
        first = first_ref[i]
        rows = jax.lax.broadcasted_iota(jnp.int32, out_ref.shape, 0)
        mask = (rows >= rs) & (rows < re)
        prev = jnp.where(first == 1, jnp.zeros_like(out_ref), out_ref[...])
        val = acc_ref[...] * wrow_ref[0]
        out_ref[...] = jnp.where(mask, val, prev)


def _grouped_ffn(xs, W1, W3, W2, w_sorted, pt, ex, rs, re, first,
                 interpret=False):
    Ts, D = xs.shape
    E, _, H = W1.shape
    L = pt.shape[0]
    ht = H // HB

    grid_spec = pltpu.PrefetchScalarGridSpec(
        num_scalar_prefetch=5,
        grid=(L, ht),
        in_specs=[
            pl.BlockSpec((TM, D), lambda i, h, pt, ex, rs, re, fi: (pt[i], 0)),
            pl.BlockSpec((1, D, HB), lambda i, h, pt, ex, rs, re, fi: (ex[i], 0, h)),
            pl.BlockSpec((1, D, HB), lambda i, h, pt, ex, rs, re, fi: (ex[i], 0, h)),
            pl.BlockSpec((1, HB, D), lambda i, h, pt, ex, rs, re, fi: (ex[i], h, 0)),
            pl.BlockSpec((1, TM, 1), lambda i, h, pt, ex, rs, re, fi: (pt[i], 0, 0)),
        ],
        out_specs=pl.BlockSpec((TM, D), lambda i, h, pt, ex, rs, re, fi: (pt[i], 0)),
        scratch_shapes=[pltpu.VMEM((TM, D), jnp.float32)],
    )
    return pl.pallas_call(
        functools.partial(_ffn_body, ht=ht),
        grid_spec=grid_spec,
        out_shape=jax.ShapeDtypeStruct((Ts, D), jnp.float32),
        compiler_params=pltpu.CompilerParams(
            dimension_semantics=("arbitrary", "arbitrary"),
        ),
        interpret=pltpu.InterpretParams() if interpret else False,
    )(pt, ex, rs, re, first, xs, W1, W3, W2,
      w_sorted.reshape(-1, TM, 1))


def _sc_combine(yw, i0, i1):
    T, D = i0.shape[0], yw.shape[1]
    NW = 32
    CB = 16
    per_w = T // NW
    n_chunks = per_w // CB
    mesh = plsc.VectorSubcoreMesh(core_axis_name="c", subcore_axis_name="s")

    @functools.partial(
        pl.kernel, mesh=mesh,
        out_type=jax.ShapeDtypeStruct((T, D), jnp.float32),
        scratch_types=[
            pltpu.VMEM((CB,), jnp.int32),
            pltpu.VMEM((CB,), jnp.int32),
            pltpu.VMEM((CB, D), jnp.float32),
            pltpu.VMEM((CB, D), jnp.float32),
            pltpu.SemaphoreType.DMA,
            pltpu.SemaphoreType.DMA,
        ],
    )
    def k(yw_hbm, i0_hbm, i1_hbm, out_hbm, idx0_v, idx1_v, r0_v, r1_v,
          s0, s1):
        wid = lax.axis_index("s") * 2 + lax.axis_index("c")
        base = wid * per_w

        def chunk(ci, carry):
            tb = base + ci * CB
            pltpu.sync_copy(i0_hbm.at[pl.ds(tb, CB)], idx0_v)
            pltpu.sync_copy(i1_hbm.at[pl.ds(tb, CB)], idx1_v)
            c0 = pltpu.async_copy(yw_hbm.at[idx0_v], r0_v, s0)
            c1 = pltpu.async_copy(yw_hbm.at[idx1_v], r1_v, s1)
            c0.wait()
            c1.wait()
            for j in range(CB):
                def col(cj, c2):
                    sl = pl.ds(cj * 16, 16)
                    r0_v[j, sl] = r0_v[j, sl] + r1_v[j, sl]
                    return c2
                lax.fori_loop(0, D // 16, col, 0, unroll=4)
            pltpu.sync_copy(r0_v, out_hbm.at[pl.ds(tb, CB)])
            return carry

        lax.fori_loop(0, n_chunks, chunk, 0)

    return k(yw, i0, i1)


def _sc_dispatch(xf, tok_ids):
    S = tok_ids.shape[0]
    D = xf.shape[1]
    NW = 32
    CB = 16
    per_w = S // NW
    n_pairs = per_w // (2 * CB)
    mesh = plsc.VectorSubcoreMesh(core_axis_name="c", subcore_axis_name="s")

    @functools.partial(
        pl.kernel, mesh=mesh,
        out_type=jax.ShapeDtypeStruct((S, D), jnp.float32),
        scratch_types=[
            pltpu.VMEM((CB,), jnp.int32),
            pltpu.VMEM((CB,), jnp.int32),
            pltpu.VMEM((CB, D), jnp.float32),
            pltpu.VMEM((CB, D), jnp.float32),
            pltpu.SemaphoreType.DMA,
            pltpu.SemaphoreType.DMA,
        ],
    )
    def k(xf_hbm, ids_hbm, out_hbm, ia_v, ib_v, ra_v, rb_v, sa, sb):
        wid = lax.axis_index("s") * 2 + lax.axis_index("c")
        base = wid * per_w

        def pair(pi, carry):
            ta = base + pi * (2 * CB)
            tb = ta + CB
            pltpu.sync_copy(ids_hbm.at[pl.ds(ta, CB)], ia_v)
            ca = pltpu.async_copy(xf_hbm.at[ia_v], ra_v, sa)
            pltpu.sync_copy(ids_hbm.at[pl.ds(tb, CB)], ib_v)
            cb = pltpu.async_copy(xf_hbm.at[ib_v], rb_v, sb)
            ca.wait()
            pltpu.sync_copy(ra_v, out_hbm.at[pl.ds(ta, CB)])
            cb.wait()
            pltpu.sync_copy(rb_v, out_hbm.at[pl.ds(tb, CB)])
            return carry

        lax.fori_loop(0, n_pairs, pair, 0)

    return k(xf, tok_ids)


def _tile_metadata(starts, ends, num_tiles, L):
    E = starts.shape[0]
    m = jnp.arange(num_tiles, dtype=jnp.int32)[:, None]
    lo, hi = m * TM, (m + 1) * TM
    st = starts[None, :].astype(jnp.int32)
    en = ends[None, :].astype(jnp.int32)
    act = (st < hi) & (en > lo)
    rs = jnp.clip(st - lo, 0, TM)
    re = jnp.clip(en - lo, 0, TM)
    ex = jnp.broadcast_to(jnp.arange(E, dtype=jnp.int32)[None, :], act.shape)
    pt = jnp.broadcast_to(m, act.shape)

    actf = act.reshape(-1)
    pos = jnp.where(actf, jnp.cumsum(actf) - 1, L + 1)
    n_real = jnp.sum(actf.astype(jnp.int32))

    def place(v):
        a = jnp.zeros((L,), jnp.int32).at[pos].set(
            v.reshape(-1).astype(jnp.int32), mode="drop")
        sel = jnp.minimum(jnp.arange(L), n_real - 1)
        return a[sel]

    pt_a, ex_a, rs_a, re_a = place(pt), place(ex), place(rs), place(re)
    first_a = (rs_a == 0).astype(jnp.int32)
    return pt_a, ex_a, rs_a, re_a, first_a


def kernel(x, Wr, W1, W3, W2, interpret=False):
    b, s, d = x.shape
    xf = x.reshape(-1, d)
    T = xf.shape[0]
    E = Wr.shape[1]
    K = 2

    logits = xf @ Wr
    probs = jax.nn.softmax(logits, axis=-1)
    topw, topi = jax.lax.top_k(probs, K)
    wts = topw / jnp.sum(topw, axis=-1, keepdims=True)

    counts = jnp.bincount(topi.reshape(-1), length=E)
    aux_loss = E * jnp.sum((counts.astype(jnp.float32) / (T * K))
                           * probs.mean(axis=0))

    Ts = T * K
    e_flat = topi.reshape(-1)
    sort_idx = jnp.argsort(e_flat, stable=True)
    tok_ids = (sort_idx // K).astype(jnp.int32)
    w_sorted = wts.reshape(-1)[sort_idx]
    inv = jnp.zeros((Ts,), jnp.int32).at[sort_idx].set(
        jnp.arange(Ts, dtype=jnp.int32))
    inv = inv.reshape(T, K)

    starts = jnp.cumsum(counts) - counts
    ends = starts + counts
    M = Ts // TM
    L = M + E - 1
    pt, ex, rs, re, first = _tile_metadata(starts, ends, M, L)

    if interpret:
        xs = jnp.take(xf, tok_ids, axis=0)
    else:
        xs = _sc_dispatch(xf, tok_ids)
    yw = _grouped_ffn(xs, W1, W3, W2, w_sorted, pt, ex, rs, re, first,
                      interpret=interpret)
    if interpret:
        out = yw[inv[:, 0]] + yw[inv[:, 1]]
    else:
        out = _sc_combine(yw, inv[:, 0] + 0, inv[:, 1] + 0)
    return out.reshape(b, s, d), aux_loss


# speedup vs baseline: 1.4415x; 1.4415x over previous
import functools

import jax
import jax.numpy as jnp
from jax import lax
from jax.experimental import pallas as pl
from jax.experimental.pallas import tpu as pltpu
from jax.experimental.pallas import tpu_sc as plsc

TM = 512
HB = 512
HC = 256


def _ffn_body(pt_ref, ex_ref, rs_ref, re_ref, first_ref,
              xs_ref, w1_ref, w3_ref, w2_ref, wrow_ref, out_ref, acc_ref, *,
              ht):
    h = pl.program_id(1)
    x = xs_ref[...]
    upd = None
    for c in range(0, HB, HC):
        g = jnp.dot(x, w1_ref[0, :, c:c + HC],
                    preferred_element_type=jnp.float32)
        u = jnp.dot(x, w3_ref[0, :, c:c + HC],
                    preferred_element_type=jnp.float32)
        mid = g * jax.nn.sigmoid(g) * u
        d = jnp.dot(mid, w2_ref[0, c:c + HC, :],
                    preferred_element_type=jnp.float32)
        upd = d if upd is None else upd + d

    @pl.when(h == 0)
    def _():
        acc_ref[...] = upd

    @pl.when(h != 0)
    def _():
        acc_ref[...] += upd

    @pl.when(h == ht - 1)
    def _():
        i = pl.program_id(0)
        rs = rs_ref[i]
        re = re_ref[i]
        first = first_ref[i]
        rows = jax.lax.broadcasted_iota(jnp.int32, out_ref.shape, 0)
        mask = (rows >= rs) & (rows < re)
        prev = jnp.where(first == 1, jnp.zeros_like(out_ref), out_ref[...])
        val = acc_ref[...] * wrow_ref[0]
        out_ref[...] = jnp.where(mask, val, prev)


def _grouped_ffn(xs, W1, W3, W2, w_sorted, pt, ex, rs, re, first,
                 interpret=False):
    Ts, D = xs.shape
    E, _, H = W1.shape
    L = pt.shape[0]
    ht = H // HB

    grid_spec = pltpu.PrefetchScalarGridSpec(
        num_scalar_prefetch=5,
        grid=(L, ht),
        in_specs=[
            pl.BlockSpec((TM, D), lambda i, h, pt, ex, rs, re, fi: (pt[i], 0)),
            pl.BlockSpec((1, D, HB), lambda i, h, pt, ex, rs, re, fi: (ex[i], 0, h)),
            pl.BlockSpec((1, D, HB), lambda i, h, pt, ex, rs, re, fi: (ex[i], 0, h)),
            pl.BlockSpec((1, HB, D), lambda i, h, pt, ex, rs, re, fi: (ex[i], h, 0)),
            pl.BlockSpec((1, TM, 1), lambda i, h, pt, ex, rs, re, fi: (pt[i], 0, 0)),
        ],
        out_specs=pl.BlockSpec((TM, D), lambda i, h, pt, ex, rs, re, fi: (pt[i], 0)),
        scratch_shapes=[pltpu.VMEM((TM, D), jnp.float32)],
    )
    return pl.pallas_call(
        functools.partial(_ffn_body, ht=ht),
        grid_spec=grid_spec,
        out_shape=jax.ShapeDtypeStruct((Ts, D), jnp.float32),
        compiler_params=pltpu.CompilerParams(
            dimension_semantics=("arbitrary", "arbitrary"),
        ),
        interpret=pltpu.InterpretParams() if interpret else False,
    )(pt, ex, rs, re, first, xs, W1, W3, W2,
      w_sorted.reshape(-1, TM, 1))


def _sc_combine(yw, i0, i1):
    T, D = i0.shape[0], yw.shape[1]
    NW = 32
    CB = 8
    per_w = T // NW
    n_pairs = per_w // (2 * CB)
    mesh = plsc.VectorSubcoreMesh(core_axis_name="c", subcore_axis_name="s")

    @functools.partial(
        pl.kernel, mesh=mesh,
        out_type=jax.ShapeDtypeStruct((T, D), jnp.float32),
        scratch_types=[
            pltpu.VMEM((CB,), jnp.int32),
            pltpu.VMEM((CB,), jnp.int32),
            pltpu.VMEM((CB,), jnp.int32),
            pltpu.VMEM((CB,), jnp.int32),
            pltpu.VMEM((CB, D), jnp.float32),
            pltpu.VMEM((CB, D), jnp.float32),
            pltpu.VMEM((CB, D), jnp.float32),
            pltpu.VMEM((CB, D), jnp.float32),
            pltpu.SemaphoreType.DMA,
            pltpu.SemaphoreType.DMA,
            pltpu.SemaphoreType.DMA,
            pltpu.SemaphoreType.DMA,
        ],
    )
    def k(yw_hbm, i0_hbm, i1_hbm, out_hbm, ia0_v, ia1_v, ib0_v, ib1_v,
          ra0_v, ra1_v, rb0_v, rb1_v, sa0, sa1, sb0, sb1):
        wid = lax.axis_index("s") * 2 + lax.axis_index("c")
        base = wid * per_w

        def add_rows(r0_v, r1_v):
            for j in range(CB):
                def col(cj, c2):
                    sl = pl.ds(cj * 16, 16)
                    r0_v[j, sl] = r0_v[j, sl] + r1_v[j, sl]
                    return c2
                lax.fori_loop(0, D // 16, col, 0, unroll=8)

        def pair(pi, carry):
            ta = base + pi * (2 * CB)
            tb = ta + CB
            pltpu.sync_copy(i0_hbm.at[pl.ds(ta, CB)], ia0_v)
            pltpu.sync_copy(i1_hbm.at[pl.ds(ta, CB)], ia1_v)
            ca0 = pltpu.async_copy(yw_hbm.at[ia0_v], ra0_v, sa0)
            ca1 = pltpu.async_copy(yw_hbm.at[ia1_v], ra1_v, sa1)
            pltpu.sync_copy(i0_hbm.at[pl.ds(tb, CB)], ib0_v)
            pltpu.sync_copy(i1_hbm.at[pl.ds(tb, CB)], ib1_v)
            cb0 = pltpu.async_copy(yw_hbm.at[ib0_v], rb0_v, sb0)
            cb1 = pltpu.async_copy(yw_hbm.at[ib1_v], rb1_v, sb1)
            ca0.wait()
            ca1.wait()
            add_rows(ra0_v, ra1_v)
            pltpu.sync_copy(ra0_v, out_hbm.at[pl.ds(ta, CB)])
            cb0.wait()
            cb1.wait()
            add_rows(rb0_v, rb1_v)
            pltpu.sync_copy(rb0_v, out_hbm.at[pl.ds(tb, CB)])
            return carry

        lax.fori_loop(0, n_pairs, pair, 0)

    return k(yw, i0, i1)


def _sc_dispatch(xf, tok_ids):
    S = tok_ids.shape[0]
    D = xf.shape[1]
    NW = 32
    CB = 16
    per_w = S // NW
    n_pairs = per_w // (2 * CB)
    mesh = plsc.VectorSubcoreMesh(core_axis_name="c", subcore_axis_name="s")

    @functools.partial(
        pl.kernel, mesh=mesh,
        out_type=jax.ShapeDtypeStruct((S, D), jnp.float32),
        scratch_types=[
            pltpu.VMEM((CB,), jnp.int32),
            pltpu.VMEM((CB,), jnp.int32),
            pltpu.VMEM((CB, D), jnp.float32),
            pltpu.VMEM((CB, D), jnp.float32),
            pltpu.SemaphoreType.DMA,
            pltpu.SemaphoreType.DMA,
        ],
    )
    def k(xf_hbm, ids_hbm, out_hbm, ia_v, ib_v, ra_v, rb_v, sa, sb):
        wid = lax.axis_index("s") * 2 + lax.axis_index("c")
        base = wid * per_w

        def pair(pi, carry):
            ta = base + pi * (2 * CB)
            tb = ta + CB
            pltpu.sync_copy(ids_hbm.at[pl.ds(ta, CB)], ia_v)
            ca = pltpu.async_copy(xf_hbm.at[ia_v], ra_v, sa)
            pltpu.sync_copy(ids_hbm.at[pl.ds(tb, CB)], ib_v)
            cb = pltpu.async_copy(xf_hbm.at[ib_v], rb_v, sb)
            ca.wait()
            pltpu.sync_copy(ra_v, out_hbm.at[pl.ds(ta, CB)])
            cb.wait()
            pltpu.sync_copy(rb_v, out_hbm.at[pl.ds(tb, CB)])
            return carry

        lax.fori_loop(0, n_pairs, pair, 0)

    return k(xf, tok_ids)


def _tile_metadata(starts, ends, num_tiles, L):
    E = starts.shape[0]
    m = jnp.arange(num_tiles, dtype=jnp.int32)[:, None]
    lo, hi = m * TM, (m + 1) * TM
    st = starts[None, :].astype(jnp.int32)
    en = ends[None, :].astype(jnp.int32)
    act = (st < hi) & (en > lo)
    rs = jnp.clip(st - lo, 0, TM)
    re = jnp.clip(en - lo, 0, TM)
    ex = jnp.broadcast_to(jnp.arange(E, dtype=jnp.int32)[None, :], act.shape)
    pt = jnp.broadcast_to(m, act.shape)

    actf = act.reshape(-1)
    pos = jnp.where(actf, jnp.cumsum(actf) - 1, L + 1)
    n_real = jnp.sum(actf.astype(jnp.int32))

    def place(v):
        a = jnp.zeros((L,), jnp.int32).at[pos].set(
            v.reshape(-1).astype(jnp.int32), mode="drop")
        sel = jnp.minimum(jnp.arange(L), n_real - 1)
        return a[sel]

    pt_a, ex_a, rs_a, re_a = place(pt), place(ex), place(rs), place(re)
    first_a = (rs_a == 0).astype(jnp.int32)
    return pt_a, ex_a, rs_a, re_a, first_a


def kernel(x, Wr, W1, W3, W2, interpret=False):
    b, s, d = x.shape
    xf = x.reshape(-1, d)
    T = xf.shape[0]
    E = Wr.shape[1]
    K = 2

    logits = xf @ Wr
    probs = jax.nn.softmax(logits, axis=-1)
    topw, topi = jax.lax.top_k(probs, K)
    wts = topw / jnp.sum(topw, axis=-1, keepdims=True)

    counts = jnp.bincount(topi.reshape(-1), length=E)
    aux_loss = E * jnp.sum((counts.astype(jnp.float32) / (T * K))
                           * probs.mean(axis=0))

    Ts = T * K
    e_flat = topi.reshape(-1)
    sort_idx = jnp.argsort(e_flat, stable=True)
    tok_ids = (sort_idx // K).astype(jnp.int32)
    w_sorted = wts.reshape(-1)[sort_idx]
    inv = jnp.zeros((Ts,), jnp.int32).at[sort_idx].set(
        jnp.arange(Ts, dtype=jnp.int32))
    inv = inv.reshape(T, K)

    starts = jnp.cumsum(counts) - counts
    ends = starts + counts
    M = Ts // TM
    L = M + E - 1
    pt, ex, rs, re, first = _tile_metadata(starts, ends, M, L)

    if interpret:
        xs = jnp.take(xf, tok_ids, axis=0)
    else:
        xs = _sc_dispatch(xf, tok_ids)
    yw = _grouped_ffn(xs, W1, W3, W2, w_sorted, pt, ex, rs, re, first,
                      interpret=interpret)
    if interpret:
        out = yw[inv[:, 0]] + yw[inv[:, 1]]
    else:
        out = _sc_combine(yw, inv[:, 0] + 0, inv[:, 1] + 0)
    return out.reshape(b, s, d), aux_loss


# speedup vs baseline: 1.4611x; 1.0136x over previous
import functools

import jax
import jax.numpy as jnp
from jax import lax
from jax.experimental import pallas as pl
from jax.experimental.pallas import tpu as pltpu
from jax.experimental.pallas import tpu_sc as plsc

TM = 512
HB = 512
HC = 256


def _ffn_body(pt_ref, ex_ref, rs_ref, re_ref, first_ref,
              xs_ref, w1_ref, w3_ref, w2_ref, wrow_ref, out_ref, acc_ref, *,
              ht):
    h = pl.program_id(1)
    x = xs_ref[...]
    upd = None
    for c in range(0, HB, HC):
        g = jnp.dot(x, w1_ref[0, :, c:c + HC],
                    preferred_element_type=jnp.float32)
        u = jnp.dot(x, w3_ref[0, :, c:c + HC],
                    preferred_element_type=jnp.float32)
        mid = g * jax.nn.sigmoid(g) * u
        d = jnp.dot(mid, w2_ref[0, c:c + HC, :],
                    preferred_element_type=jnp.float32)
        upd = d if upd is None else upd + d

    @pl.when(h == 0)
    def _():
        acc_ref[...] = upd

    @pl.when(h != 0)
    def _():
        acc_ref[...] += upd

    @pl.when(h == ht - 1)
    def _():
        i = pl.program_id(0)
        rs = rs_ref[i]
        re = re_ref[i]
        first = first_ref[i]
        rows = jax.lax.broadcasted_iota(jnp.int32, out_ref.shape, 0)
        mask = (rows >= rs) & (rows < re)
        prev = jnp.where(first == 1, jnp.zeros_like(out_ref), out_ref[...])
        val = acc_ref[...] * wrow_ref[0]
        out_ref[...] = jnp.where(mask, val, prev)


def _grouped_ffn(xs, W1, W3, W2, w_sorted, pt, ex, rs, re, first,
                 interpret=False):
    Ts, D = xs.shape
    E, _, H = W1.shape
    L = pt.shape[0]
    ht = H // HB

    grid_spec = pltpu.PrefetchScalarGridSpec(
        num_scalar_prefetch=5,
        grid=(L, ht),
        in_specs=[
            pl.BlockSpec((TM, D), lambda i, h, pt, ex, rs, re, fi: (pt[i], 0)),
            pl.BlockSpec((1, D, HB), lambda i, h, pt, ex, rs, re, fi: (ex[i], 0, h)),
            pl.BlockSpec((1, D, HB), lambda i, h, pt, ex, rs, re, fi: (ex[i], 0, h)),
            pl.BlockSpec((1, HB, D), lambda i, h, pt, ex, rs, re, fi: (ex[i], h, 0)),
            pl.BlockSpec((1, TM, 1), lambda i, h, pt, ex, rs, re, fi: (pt[i], 0, 0)),
        ],
        out_specs=pl.BlockSpec((TM, D), lambda i, h, pt, ex, rs, re, fi: (pt[i], 0)),
        scratch_shapes=[pltpu.VMEM((TM, D), jnp.float32)],
    )
    return pl.pallas_call(
        functools.partial(_ffn_body, ht=ht),
        grid_spec=grid_spec,
        out_shape=jax.ShapeDtypeStruct((Ts, D), jnp.float32),
        compiler_params=pltpu.CompilerParams(
            dimension_semantics=("arbitrary", "arbitrary"),
        ),
        interpret=pltpu.InterpretParams() if interpret else False,
    )(pt, ex, rs, re, first, xs, W1, W3, W2,
      w_sorted.reshape(-1, TM, 1))


def _sc_combine(yw, i0, i1):
    T, D = i0.shape[0], yw.shape[1]
    NW = 32
    CB = 16
    per_w = T // NW
    n_chunks = per_w // CB
    mesh = plsc.VectorSubcoreMesh(core_axis_name="c", subcore_axis_name="s")

    @functools.partial(
        pl.kernel, mesh=mesh,
        out_type=jax.ShapeDtypeStruct((T, D), jnp.float32),
        scratch_types=[
            pltpu.VMEM((CB,), jnp.int32),
            pltpu.VMEM((CB,), jnp.int32),
            pltpu.VMEM((CB, D), jnp.float32),
            pltpu.VMEM((CB, D), jnp.float32),
            pltpu.SemaphoreType.DMA,
            pltpu.SemaphoreType.DMA,
        ],
    )
    def k(yw_hbm, i0_hbm, i1_hbm, out_hbm, idx0_v, idx1_v, r0_v, r1_v,
          s0, s1):
        wid = lax.axis_index("s") * 2 + lax.axis_index("c")
        base = wid * per_w

        def chunk(ci, carry):
            tb = base + ci * CB
            pltpu.sync_copy(i0_hbm.at[pl.ds(tb, CB)], idx0_v)
            pltpu.sync_copy(i1_hbm.at[pl.ds(tb, CB)], idx1_v)
            c0 = pltpu.async_copy(yw_hbm.at[idx0_v], r0_v, s0)
            c1 = pltpu.async_copy(yw_hbm.at[idx1_v], r1_v, s1)
            c0.wait()
            c1.wait()
            for j in range(CB):
                def col(cj, c2):
                    sl = pl.ds(cj * 16, 16)
                    r0_v[j, sl] = r0_v[j, sl] + r1_v[j, sl]
                    return c2
                lax.fori_loop(0, D // 16, col, 0, unroll=8)
            pltpu.sync_copy(r0_v, out_hbm.at[pl.ds(tb, CB)])
            return carry

        lax.fori_loop(0, n_chunks, chunk, 0)

    return k(yw, i0, i1)


def _sc_dispatch(xf, tok_ids):
    S = tok_ids.shape[0]
    D = xf.shape[1]
    NW = 32
    CB = 16
    per_w = S // NW
    n_pairs = per_w // (2 * CB)
    mesh = plsc.VectorSubcoreMesh(core_axis_name="c", subcore_axis_name="s")

    @functools.partial(
        pl.kernel, mesh=mesh,
        out_type=jax.ShapeDtypeStruct((S, D), jnp.float32),
        scratch_types=[
            pltpu.VMEM((CB,), jnp.int32),
            pltpu.VMEM((CB,), jnp.int32),
            pltpu.VMEM((CB, D), jnp.float32),
            pltpu.VMEM((CB, D), jnp.float32),
            pltpu.SemaphoreType.DMA,
            pltpu.SemaphoreType.DMA,
        ],
    )
    def k(xf_hbm, ids_hbm, out_hbm, ia_v, ib_v, ra_v, rb_v, sa, sb):
        wid = lax.axis_index("s") * 2 + lax.axis_index("c")
        base = wid * per_w

        def pair(pi, carry):
            ta = base + pi * (2 * CB)
            tb = ta + CB
            pltpu.sync_copy(ids_hbm.at[pl.ds(ta, CB)], ia_v)
            ca = pltpu.async_copy(xf_hbm.at[ia_v], ra_v, sa)
            pltpu.sync_copy(ids_hbm.at[pl.ds(tb, CB)], ib_v)
            cb = pltpu.async_copy(xf_hbm.at[ib_v], rb_v, sb)
            ca.wait()
            pltpu.sync_copy(ra_v, out_hbm.at[pl.ds(ta, CB)])
            cb.wait()
            pltpu.sync_copy(rb_v, out_hbm.at[pl.ds(tb, CB)])
            return carry

        lax.fori_loop(0, n_pairs, pair, 0)

    return k(xf, tok_ids)


def _tile_metadata(starts, ends, num_tiles, L):
    E = starts.shape[0]
    m = jnp.arange(num_tiles, dtype=jnp.int32)[:, None]
    lo, hi = m * TM, (m + 1) * TM
    st = starts[None, :].astype(jnp.int32)
    en = ends[None, :].astype(jnp.int32)
    act = (st < hi) & (en > lo)
    rs = jnp.clip(st - lo, 0, TM)
    re = jnp.clip(en - lo, 0, TM)
    ex = jnp.broadcast_to(jnp.arange(E, dtype=jnp.int32)[None, :], act.shape)
    pt = jnp.broadcast_to(m, act.shape)

    actf = act.reshape(-1)
    pos = jnp.where(actf, jnp.cumsum(actf) - 1, L + 1)
    n_real = jnp.sum(actf.astype(jnp.int32))

    def place(v):
        a = jnp.zeros((L,), jnp.int32).at[pos].set(
            v.reshape(-1).astype(jnp.int32), mode="drop")
        sel = jnp.minimum(jnp.arange(L), n_real - 1)
        return a[sel]

    pt_a, ex_a, rs_a, re_a = place(pt), place(ex), place(rs), place(re)
    first_a = (rs_a == 0).astype(jnp.int32)
    return pt_a, ex_a, rs_a, re_a, first_a


def kernel(x, Wr, W1, W3, W2, interpret=False):
    b, s, d = x.shape
    xf = x.reshape(-1, d)
    T = xf.shape[0]
    E = Wr.shape[1]
    K = 2

    logits = xf @ Wr
    probs = jax.nn.softmax(logits, axis=-1)
    topw, topi = jax.lax.top_k(probs, K)
    wts = topw / jnp.sum(topw, axis=-1, keepdims=True)

    counts = jnp.bincount(topi.reshape(-1), length=E)
    aux_loss = E * jnp.sum((counts.astype(jnp.float32) / (T * K))
                           * probs.mean(axis=0))

    Ts = T * K
    e_flat = topi.reshape(-1)
    sort_idx = jnp.argsort(e_flat, stable=True)
    tok_ids = (sort_idx // K).astype(jnp.int32)
    w_sorted = wts.reshape(-1)[sort_idx]
    inv = jnp.zeros((Ts,), jnp.int32).at[sort_idx].set(
        jnp.arange(Ts, dtype=jnp.int32))
    inv = inv.reshape(T, K)

    starts = jnp.cumsum(counts) - counts
    ends = starts + counts
    M = Ts // TM
    L = M + E - 1
    pt, ex, rs, re, first = _tile_metadata(starts, ends, M, L)

    if interpret:
        xs = jnp.take(xf, tok_ids, axis=0)
    else:
        xs = _sc_dispatch(xf, tok_ids)
    yw = _grouped_ffn(xs, W1, W3, W2, w_sorted, pt, ex, rs, re, first,
                      interpret=interpret)
    if interpret:
        out = yw[inv[:, 0]] + yw[inv[:, 1]]
    else:
        out = _sc_combine(yw, inv[:, 0] + 0, inv[:, 1] + 0)
    return out.reshape(b, s, d), aux_loss


# speedup vs baseline: 1.4622x; 1.0007x over previous
import functools

import jax
import jax.numpy as jnp
from jax import lax
from jax.experimental import pallas as pl
from jax.experimental.pallas import tpu as pltpu
from jax.experimental.pallas import tpu_sc as plsc

TM = 512
HB = 512
HC = 256


def _ffn_body(pt_ref, ex_ref, rs_ref, re_ref, first_ref,
              xs_ref, w1_ref, w3_ref, w2_ref, wrow_ref, out_ref, acc_ref, *,
              ht):
    h = pl.program_id(1)
    x = xs_ref[...]
    upd = None
    for c in range(0, HB, HC):
        g = jnp.dot(x, w1_ref[0, :, c:c + HC],
                    preferred_element_type=jnp.float32)
        u = jnp.dot(x, w3_ref[0, :, c:c + HC],
                    preferred_element_type=jnp.float32)
        mid = g * jax.nn.sigmoid(g) * u
        d = jnp.dot(mid, w2_ref[0, c:c + HC, :],
                    preferred_element_type=jnp.float32)
        upd = d if upd is None else upd + d

    @pl.when(h == 0)
    def _():
        acc_ref[...] = upd

    @pl.when(h != 0)
    def _():
        acc_ref[...] += upd

    @pl.when(h == ht - 1)
    def _():
        i = pl.program_id(0)
        rs = rs_ref[i]
        re = re_ref[i]
        first = first_ref[i]
        rows = jax.lax.broadcasted_iota(jnp.int32, out_ref.shape, 0)
        mask = (rows >= rs) & (rows < re)
        prev = jnp.where(first == 1, jnp.zeros_like(out_ref), out_ref[...])
        val = acc_ref[...] * wrow_ref[0]
        out_ref[...] = jnp.where(mask, val, prev)


def _grouped_ffn(xs, W1, W3, W2, w_sorted, pt, ex, rs, re, first):
    Ts, D = xs.shape
    E, _, H = W1.shape
    L = pt.shape[0]
    ht = H // HB

    grid_spec = pltpu.PrefetchScalarGridSpec(
        num_scalar_prefetch=5,
        grid=(L, ht),
        in_specs=[
            pl.BlockSpec((TM, D), lambda i, h, pt, ex, rs, re, fi: (pt[i], 0)),
            pl.BlockSpec((1, D, HB), lambda i, h, pt, ex, rs, re, fi: (ex[i], 0, h)),
            pl.BlockSpec((1, D, HB), lambda i, h, pt, ex, rs, re, fi: (ex[i], 0, h)),
            pl.BlockSpec((1, HB, D), lambda i, h, pt, ex, rs, re, fi: (ex[i], h, 0)),
            pl.BlockSpec((1, TM, 1), lambda i, h, pt, ex, rs, re, fi: (pt[i], 0, 0)),
        ],
        out_specs=pl.BlockSpec((TM, D), lambda i, h, pt, ex, rs, re, fi: (pt[i], 0)),
        scratch_shapes=[pltpu.VMEM((TM, D), jnp.float32)],
    )
    return pl.pallas_call(
        functools.partial(_ffn_body, ht=ht),
        grid_spec=grid_spec,
        out_shape=jax.ShapeDtypeStruct((Ts, D), jnp.float32),
        compiler_params=pltpu.CompilerParams(
            dimension_semantics=("arbitrary", "arbitrary"),
        ),
    )(pt, ex, rs, re, first, xs, W1, W3, W2,
      w_sorted.reshape(-1, TM, 1))


def _sc_combine(yw, i0, i1):
    T, D = i0.shape[0], yw.shape[1]
    NW = 32
    CB = 16
    per_w = T // NW
    n_chunks = per_w // CB
    mesh = plsc.VectorSubcoreMesh(core_axis_name="c", subcore_axis_name="s")

    @functools.partial(
        pl.kernel, mesh=mesh,
        out_type=jax.ShapeDtypeStruct((T, D), jnp.float32),
        scratch_types=[
            pltpu.VMEM((CB,), jnp.int32),
            pltpu.VMEM((CB,), jnp.int32),
            pltpu.VMEM((CB, D), jnp.float32),
            pltpu.VMEM((CB, D), jnp.float32),
            pltpu.SemaphoreType.DMA,
            pltpu.SemaphoreType.DMA,
        ],
    )
    def k(yw_hbm, i0_hbm, i1_hbm, out_hbm, idx0_v, idx1_v, r0_v, r1_v,
          s0, s1):
        wid = lax.axis_index("s") * 2 + lax.axis_index("c")
        base = wid * per_w

        def chunk(ci, carry):
            tb = base + ci * CB
            pltpu.sync_copy(i0_hbm.at[pl.ds(tb, CB)], idx0_v)
            pltpu.sync_copy(i1_hbm.at[pl.ds(tb, CB)], idx1_v)
            c0 = pltpu.async_copy(yw_hbm.at[idx0_v], r0_v, s0)
            c1 = pltpu.async_copy(yw_hbm.at[idx1_v], r1_v, s1)
            c0.wait()
            c1.wait()
            for j in range(CB):
                def col(cj, c2):
                    sl = pl.ds(cj * 16, 16)
                    r0_v[j, sl] = r0_v[j, sl] + r1_v[j, sl]
                    return c2
                lax.fori_loop(0, D // 16, col, 0, unroll=8)
            pltpu.sync_copy(r0_v, out_hbm.at[pl.ds(tb, CB)])
            return carry

        lax.fori_loop(0, n_chunks, chunk, 0)

    return k(yw, i0, i1)


def _sc_dispatch(xf, tok_ids):
    S = tok_ids.shape[0]
    D = xf.shape[1]
    NW = 32
    CB = 16
    per_w = S // NW
    n_pairs = per_w // (2 * CB)
    mesh = plsc.VectorSubcoreMesh(core_axis_name="c", subcore_axis_name="s")

    @functools.partial(
        pl.kernel, mesh=mesh,
        out_type=jax.ShapeDtypeStruct((S, D), jnp.float32),
        scratch_types=[
            pltpu.VMEM((CB,), jnp.int32),
            pltpu.VMEM((CB,), jnp.int32),
            pltpu.VMEM((CB, D), jnp.float32),
            pltpu.VMEM((CB, D), jnp.float32),
            pltpu.SemaphoreType.DMA,
            pltpu.SemaphoreType.DMA,
        ],
    )
    def k(xf_hbm, ids_hbm, out_hbm, ia_v, ib_v, ra_v, rb_v, sa, sb):
        wid = lax.axis_index("s") * 2 + lax.axis_index("c")
        base = wid * per_w

        def pair(pi, carry):
            ta = base + pi * (2 * CB)
            tb = ta + CB
            pltpu.sync_copy(ids_hbm.at[pl.ds(ta, CB)], ia_v)
            ca = pltpu.async_copy(xf_hbm.at[ia_v], ra_v, sa)
            pltpu.sync_copy(ids_hbm.at[pl.ds(tb, CB)], ib_v)
            cb = pltpu.async_copy(xf_hbm.at[ib_v], rb_v, sb)
            ca.wait()
            pltpu.sync_copy(ra_v, out_hbm.at[pl.ds(ta, CB)])
            cb.wait()
            pltpu.sync_copy(rb_v, out_hbm.at[pl.ds(tb, CB)])
            return carry

        lax.fori_loop(0, n_pairs, pair, 0)

    return k(xf, tok_ids)


def _tile_metadata(starts, ends, num_tiles, L):
    E = starts.shape[0]
    m = jnp.arange(num_tiles, dtype=jnp.int32)[:, None]
    lo, hi = m * TM, (m + 1) * TM
    st = starts[None, :].astype(jnp.int32)
    en = ends[None, :].astype(jnp.int32)
    act = (st < hi) & (en > lo)
    rs = jnp.clip(st - lo, 0, TM)
    re = jnp.clip(en - lo, 0, TM)
    ex = jnp.broadcast_to(jnp.arange(E, dtype=jnp.int32)[None, :], act.shape)
    pt = jnp.broadcast_to(m, act.shape)

    actf = act.reshape(-1)
    pos = jnp.where(actf, jnp.cumsum(actf) - 1, L + 1)
    n_real = jnp.sum(actf.astype(jnp.int32))

    def place(v):
        a = jnp.zeros((L,), jnp.int32).at[pos].set(
            v.reshape(-1).astype(jnp.int32), mode="drop")
        sel = jnp.minimum(jnp.arange(L), n_real - 1)
        return a[sel]

    pt_a, ex_a, rs_a, re_a = place(pt), place(ex), place(rs), place(re)
    first_a = (rs_a == 0).astype(jnp.int32)
    return pt_a, ex_a, rs_a, re_a, first_a


def kernel(x, Wr, W1, W3, W2):
    b, s, d = x.shape
    xf = x.reshape(-1, d)
    T = xf.shape[0]
    E = Wr.shape[1]
    K = 2

    logits = xf @ Wr
    probs = jax.nn.softmax(logits, axis=-1)
    topw, topi = jax.lax.top_k(probs, K)
    wts = topw / jnp.sum(topw, axis=-1, keepdims=True)

    counts = jnp.bincount(topi.reshape(-1), length=E)
    aux_loss = E * jnp.sum((counts.astype(jnp.float32) / (T * K))
                           * probs.mean(axis=0))

    Ts = T * K
    e_flat = topi.reshape(-1)
    sort_idx = jnp.argsort(e_flat, stable=True)
    tok_ids = (sort_idx // K).astype(jnp.int32)
    w_sorted = wts.reshape(-1)[sort_idx]
    inv = jnp.zeros((Ts,), jnp.int32).at[sort_idx].set(
        jnp.arange(Ts, dtype=jnp.int32))
    inv = inv.reshape(T, K)

    starts = jnp.cumsum(counts) - counts
    ends = starts + counts
    M = Ts // TM
    L = M + E - 1
    pt, ex, rs, re, first = _tile_metadata(starts, ends, M, L)

    xs = _sc_dispatch(xf, tok_ids)
    yw = _grouped_ffn(xs, W1, W3, W2, w_sorted, pt, ex, rs, re, first)
    out = _sc_combine(yw, inv[:, 0] + 0, inv[:, 1] + 0)
    return out.reshape(b, s, d), aux_loss
